# baseline (speedup 1.0000x reference)
import jax
import jax.numpy as jnp
from jax.experimental import pallas as pl


def _body(x_ref, p_ref, o_ref):
    o_ref[...] = x_ref[...] + p_ref[...]


def kernel(encoded_patches, pos_table):
    B, P, D = encoded_patches.shape
    PB = 256
    grid = (P // PB, B)
    return pl.pallas_call(
        _body,
        grid=grid,
        in_specs=[
            pl.BlockSpec((1, PB, D), lambda i, j: (j, i, 0)),
            pl.BlockSpec((PB, D), lambda i, j: (i, 0)),
        ],
        out_specs=pl.BlockSpec((1, PB, D), lambda i, j: (j, i, 0)),
        out_shape=jax.ShapeDtypeStruct((B, P, D), encoded_patches.dtype),
    )(encoded_patches, pos_table)


# speedup vs baseline: 1.7361x; 1.7361x over previous
import jax
import jax.numpy as jnp
from jax.experimental import pallas as pl


def _body(x_ref, p_ref, o_ref):
    o_ref[...] = x_ref[...] + p_ref[...]


def kernel(encoded_patches, pos_table):
    B, P, D = encoded_patches.shape
    x2 = encoded_patches.reshape(B * P, D)
    out = pl.pallas_call(
        _body,
        grid=(B,),
        in_specs=[
            pl.BlockSpec((P, D), lambda i: (i, 0)),
            pl.BlockSpec((P, D), lambda i: (0, 0)),
        ],
        out_specs=pl.BlockSpec((P, D), lambda i: (i, 0)),
        out_shape=jax.ShapeDtypeStruct((B * P, D), encoded_patches.dtype),
    )(x2, pos_table)
    return out.reshape(B, P, D)
